# Initial kernel scaffold; baseline (speedup 1.0000x reference)
import jax
import jax.numpy as jnp
from jax.experimental import pallas as pl


def kernel(encoded_input, start_ids, end_ids, W, b):
    raise NotImplementedError("write your pallas kernel here")


# speedup vs baseline: 8.6946x; 8.6946x over previous
import functools

import jax
import jax.numpy as jnp
from jax import lax
from jax.experimental import pallas as pl
from jax.experimental.pallas import tpu as pltpu
from jax.experimental.pallas import tpu_sc as plsc

BATCH = 64
SEQ = 2048
D_IN = 1024
D_PROJ = 256

_NUM_WORKERS = 8
_ROWS_PER_W = 16


def _gather_body(table_hbm, start_hbm, end_hbm, out_hbm, ids_v, idx_v, rows_v, sem):
    c = lax.axis_index("c")
    s = lax.axis_index("s")
    wid = s * 2 + c

    @pl.when(wid < _NUM_WORKERS)
    def _():
        is_end = wid >= 4
        chunk = wid % 4
        b0 = chunk * _ROWS_PER_W

        @pl.when(is_end)
        def _():
            pltpu.sync_copy(end_hbm.at[pl.ds(b0, _ROWS_PER_W)], ids_v)

        @pl.when(jnp.logical_not(is_end))
        def _():
            pltpu.sync_copy(start_hbm.at[pl.ds(b0, _ROWS_PER_W)], ids_v)

        batch = b0 + lax.iota(jnp.int32, _ROWS_PER_W)
        idx_v[...] = ids_v[...] + batch * SEQ

        pltpu.async_copy(table_hbm.at[idx_v], rows_v, sem).wait()

        out_base = wid * _ROWS_PER_W
        pltpu.sync_copy(rows_v, out_hbm.at[pl.ds(out_base, _ROWS_PER_W)])


_gather_rows = functools.partial(
    pl.kernel,
    mesh=plsc.VectorSubcoreMesh(core_axis_name="c", subcore_axis_name="s"),
    out_type=jax.ShapeDtypeStruct((2 * BATCH, D_IN), jnp.float32),
    scratch_types=[
        pltpu.VMEM((_ROWS_PER_W,), jnp.int32),
        pltpu.VMEM((_ROWS_PER_W,), jnp.int32),
        pltpu.VMEM((_ROWS_PER_W, D_IN), jnp.float32),
        pltpu.SemaphoreType.DMA,
    ],
)(_gather_body)


def _proj_body(g_ref, w_ref, b_ref, o_ref):
    r = lax.dot_general(
        g_ref[...], w_ref[...],
        dimension_numbers=(((1,), (1,)), ((), ())),
        preferred_element_type=jnp.float32,
    )
    r = r + b_ref[...]
    o_ref[:, :D_PROJ] = r[:BATCH, :]
    o_ref[:, D_PROJ:] = r[BATCH:, :]


def kernel(encoded_input, start_ids, end_ids, W, b):
    table = encoded_input.reshape(BATCH * SEQ, D_IN)
    gathered = _gather_rows(
        table,
        start_ids.astype(jnp.int32),
        end_ids.astype(jnp.int32),
    )
    return pl.pallas_call(
        _proj_body,
        out_shape=jax.ShapeDtypeStruct((BATCH, 2 * D_PROJ), jnp.float32),
    )(gathered, W, b.reshape(1, D_PROJ))


# speedup vs baseline: 9.3293x; 1.0730x over previous
import functools

import jax
import jax.numpy as jnp
from jax import lax
from jax.experimental import pallas as pl
from jax.experimental.pallas import tpu as pltpu
from jax.experimental.pallas import tpu_sc as plsc

BATCH = 64
SEQ = 2048
D_IN = 1024
D_PROJ = 256

_NUM_WORKERS = 8
_ROWS_PER_W = 16


def _gather_body(table_hbm, start_hbm, end_hbm, out_hbm, ids_v, idx_v, rows_v, sem):
    wid = lax.axis_index("s")

    @pl.when(wid < _NUM_WORKERS)
    def _():
        is_end = wid >= 4
        chunk = wid % 4
        b0 = chunk * _ROWS_PER_W

        @pl.when(is_end)
        def _():
            pltpu.sync_copy(end_hbm.at[pl.ds(b0, _ROWS_PER_W)], ids_v)

        @pl.when(jnp.logical_not(is_end))
        def _():
            pltpu.sync_copy(start_hbm.at[pl.ds(b0, _ROWS_PER_W)], ids_v)

        batch = b0 + lax.iota(jnp.int32, _ROWS_PER_W)
        idx_v[...] = ids_v[...] + batch * SEQ

        pltpu.async_copy(table_hbm.at[idx_v], rows_v, sem).wait()

        out_base = wid * _ROWS_PER_W
        pltpu.sync_copy(rows_v, out_hbm.at[pl.ds(out_base, _ROWS_PER_W)])


_gather_rows = functools.partial(
    pl.kernel,
    mesh=plsc.VectorSubcoreMesh(core_axis_name="c", subcore_axis_name="s", num_cores=1),
    out_type=jax.ShapeDtypeStruct((2 * BATCH, D_IN), jnp.float32),
    scratch_types=[
        pltpu.VMEM((_ROWS_PER_W,), jnp.int32),
        pltpu.VMEM((_ROWS_PER_W,), jnp.int32),
        pltpu.VMEM((_ROWS_PER_W, D_IN), jnp.float32),
        pltpu.SemaphoreType.DMA,
    ],
)(_gather_body)


def _proj_body(g_ref, w_ref, b_ref, o_ref):
    r = lax.dot_general(
        g_ref[...], w_ref[...],
        dimension_numbers=(((1,), (1,)), ((), ())),
        preferred_element_type=jnp.float32,
    )
    r = r + b_ref[...]
    o_ref[:, :D_PROJ] = r[:BATCH, :]
    o_ref[:, D_PROJ:] = r[BATCH:, :]


def kernel(encoded_input, start_ids, end_ids, W, b):
    table = encoded_input.reshape(BATCH * SEQ, D_IN)
    gathered = _gather_rows(
        table,
        start_ids.astype(jnp.int32),
        end_ids.astype(jnp.int32),
    )
    return pl.pallas_call(
        _proj_body,
        out_shape=jax.ShapeDtypeStruct((BATCH, 2 * D_PROJ), jnp.float32),
    )(gathered, W, b.reshape(1, D_PROJ))


# speedup vs baseline: 9.6102x; 1.0301x over previous
import functools

import jax
import jax.numpy as jnp
from jax import lax
from jax.experimental import pallas as pl
from jax.experimental.pallas import tpu as pltpu
from jax.experimental.pallas import tpu_sc as plsc

BATCH = 64
SEQ = 2048
D_IN = 1024
D_PROJ = 256

_ROWS_PER_W = 8


def _gather_body(idx_hbm, table_hbm, out_hbm, idx_v, rows_v, sem):
    wid = lax.axis_index("s")
    base = wid * _ROWS_PER_W

    pltpu.sync_copy(idx_hbm.at[pl.ds(base, _ROWS_PER_W)], idx_v)
    pltpu.async_copy(table_hbm.at[idx_v], rows_v, sem).wait()
    pltpu.sync_copy(rows_v, out_hbm.at[pl.ds(base, _ROWS_PER_W)])


_gather_rows = functools.partial(
    pl.kernel,
    mesh=plsc.VectorSubcoreMesh(core_axis_name="c", subcore_axis_name="s", num_cores=1),
    out_type=jax.ShapeDtypeStruct((2 * BATCH, D_IN), jnp.float32),
    scratch_types=[
        pltpu.VMEM((_ROWS_PER_W,), jnp.int32),
        pltpu.VMEM((_ROWS_PER_W, D_IN), jnp.float32),
        pltpu.SemaphoreType.DMA,
    ],
)(_gather_body)


def _proj_body(g_ref, w_ref, b_ref, o_ref):
    r = lax.dot_general(
        g_ref[...], w_ref[...],
        dimension_numbers=(((1,), (1,)), ((), ())),
        preferred_element_type=jnp.float32,
    )
    r = r + b_ref[...]
    o_ref[:, :D_PROJ] = r[:BATCH, :]
    o_ref[:, D_PROJ:] = r[BATCH:, :]


def kernel(encoded_input, start_ids, end_ids, W, b):
    table = encoded_input.reshape(BATCH * SEQ, D_IN)
    offs = jnp.arange(BATCH, dtype=jnp.int32) * SEQ
    idx = jnp.concatenate(
        [start_ids.astype(jnp.int32) + offs, end_ids.astype(jnp.int32) + offs]
    )
    gathered = _gather_rows(idx, table)
    return pl.pallas_call(
        _proj_body,
        out_shape=jax.ShapeDtypeStruct((BATCH, 2 * D_PROJ), jnp.float32),
    )(gathered, W, b.reshape(1, D_PROJ))
